# baseline (speedup 1.0000x reference)
import functools

import jax
import jax.numpy as jnp
from jax import lax
from jax.experimental import pallas as pl
from jax.experimental.pallas import tpu as pltpu
from jax.experimental.pallas import tpu_sc as plsc

MAX_DEG = 64
VOCAB = MAX_DEG + 1
D = 128
N_ROWS = 100000
NUM_CORES = 2
NUM_SUBCORES = 16
NW = NUM_CORES * NUM_SUBCORES
CH = 128
NFULL = N_ROWS // CH
TAIL = N_ROWS - NFULL * CH
CPW = (NFULL + 1 + NW - 1) // NW
PAD_N = CPW * NW * CH
NBUF = 4


def _prep_body(win_ref, wout_ref, ind_ref, outd_ref, wsum_ref, idx_ref):
    win = win_ref[...]
    wout = wout_ref[...]
    wsum_ref[...] = win[:, None, :] + wout[None, :, :]
    a = jnp.clip(ind_ref[...], 0, MAX_DEG)
    b = jnp.clip(outd_ref[...], 0, MAX_DEG)
    idx_ref[...] = a * VOCAB + b


_mesh = plsc.VectorSubcoreMesh(core_axis_name="c", subcore_axis_name="s")


@functools.partial(
    pl.kernel,
    mesh=_mesh,
    out_type=jax.ShapeDtypeStruct((N_ROWS, D), jnp.float32),
    scratch_types=[
        pltpu.VMEM((2, CH), jnp.int32),
        pltpu.VMEM((NBUF, CH, D), jnp.float32),
        pltpu.SemaphoreType.DMA,
        pltpu.SemaphoreType.DMA,
        pltpu.SemaphoreType.DMA((NBUF,)),
    ],
)
def _sc_gather(wsum_hbm, idx_hbm, out_hbm, idx_v, rows_v, sem_i, sem_g, sem_w):
    wid = lax.axis_index("s") * NUM_CORES + lax.axis_index("c")

    def gbase(c):
        return (c * NW + wid) * CH

    pltpu.sync_copy(idx_hbm.at[pl.ds(gbase(0), CH)], idx_v.at[0])
    gather = pltpu.async_copy(wsum_hbm.at[idx_v.at[0]], rows_v.at[0], sem_g)
    idx_pend = pltpu.async_copy(idx_hbm.at[pl.ds(gbase(1), CH)], idx_v.at[1],
                                sem_i)

    writes = {}
    for c in range(CPW):
        b = c % NBUF
        g = c * NW + wid
        gather.wait()
        if c < CPW - 1:
            writes[c] = pltpu.async_copy(
                rows_v.at[b], out_hbm.at[pl.ds(g * CH, CH)], sem_w.at[b])
            nc = c + 1
            nb = nc % NBUF
            if nc - NBUF >= 0:
                writes[nc - NBUF].wait()
            idx_pend.wait()
            gather = pltpu.async_copy(
                wsum_hbm.at[idx_v.at[nc % 2]], rows_v.at[nb], sem_g)
            if nc + 1 < CPW:
                idx_pend = pltpu.async_copy(
                    idx_hbm.at[pl.ds(gbase(nc + 1), CH)],
                    idx_v.at[(nc + 1) % 2], sem_i)
        else:
            @pl.when(g < NFULL)
            def _():
                pltpu.async_copy(rows_v.at[b],
                                 out_hbm.at[pl.ds(g * CH, CH)],
                                 sem_w.at[b]).wait()

            @pl.when(g == NFULL)
            def _():
                pltpu.async_copy(rows_v.at[b, pl.ds(0, TAIL)],
                                 out_hbm.at[pl.ds(g * CH, TAIL)],
                                 sem_w.at[b]).wait()
    for k in range(max(0, CPW - NBUF), CPW - 1):
        writes[k].wait()


def kernel(in_degree, out_degree, W_in, W_out):
    pad = PAD_N - N_ROWS
    rows = PAD_N // D
    ind = jnp.pad(in_degree.astype(jnp.int32), (0, pad)).reshape(rows, D)
    outd = jnp.pad(out_degree.astype(jnp.int32), (0, pad)).reshape(rows, D)
    wsum, idxc = pl.pallas_call(
        _prep_body,
        out_shape=[
            jax.ShapeDtypeStruct((VOCAB, VOCAB, D), jnp.float32),
            jax.ShapeDtypeStruct((rows, D), jnp.int32),
        ],
    )(W_in, W_out, ind, outd)
    return _sc_gather(wsum.reshape(VOCAB * VOCAB, D), idxc.reshape(PAD_N))
